# Initial kernel scaffold; baseline (speedup 1.0000x reference)
import jax
import jax.numpy as jnp
from jax.experimental import pallas as pl


def kernel(token_ids, table):
    raise NotImplementedError("write your pallas kernel here")


# speedup vs baseline: 1.5962x; 1.5962x over previous
import functools

import jax
import jax.numpy as jnp
from jax import lax
from jax.experimental import pallas as pl
from jax.experimental.pallas import tpu as pltpu
from jax.experimental.pallas import tpu_sc as plsc

VOCAB = 1000000
DIM = 64
NUM_BITS = 16
BITS_BEGIN = 8
B = 16384
L = 50

N = B * L
NC, NS = 2, 16
NW = NC * NS
PER_W = N // NW
CHUNK = 128
NCHUNK = PER_W // CHUNK


def _make_kernel():
  mesh = plsc.VectorSubcoreMesh(core_axis_name="c", subcore_axis_name="s")

  @functools.partial(
      pl.kernel,
      mesh=mesh,
      compiler_params=pltpu.CompilerParams(use_tc_tiling_on_sc=False),
      out_type=jax.ShapeDtypeStruct((N, DIM), jnp.float32),
      scratch_types=[
          pltpu.VMEM((CHUNK,), jnp.int32),
          pltpu.VMEM((CHUNK, DIM), jnp.float32),
          pltpu.SemaphoreType.DMA,
      ],
  )
  def embed_kernel(ids_hbm, table_hbm, out_hbm, idx_v, rows_v, sem):
    wid = lax.axis_index("s") * NC + lax.axis_index("c")
    w_base = wid * PER_W

    def chunk_body(c, carry):
      base = w_base + c * CHUNK
      pltpu.sync_copy(ids_hbm.at[pl.ds(base, CHUNK)], idx_v)
      pltpu.async_copy(table_hbm.at[idx_v], rows_v, sem).wait()
      lanes = jnp.arange(NUM_BITS, dtype=jnp.int32)
      for g in range(CHUNK // 16):
        ids16 = idx_v[pl.ds(g * 16, 16)]
        dnums = lax.GatherDimensionNumbers(
            offset_dims=(), collapsed_slice_dims=(0,), start_index_map=(0,))
        for r in range(16):
          sel = jnp.full((16, 1), r, dtype=jnp.int32)
          idv = lax.gather(ids16, sel, dnums, (1,),
                           mode=lax.GatherScatterMode.PROMISE_IN_BOUNDS)
          bits = ((idv >> lanes) & 1).astype(jnp.float32)
          rows_v[g * 16 + r, pl.ds(BITS_BEGIN, NUM_BITS)] = bits
      pltpu.sync_copy(rows_v, out_hbm.at[pl.ds(base, CHUNK)])
      return carry

    lax.fori_loop(0, NCHUNK, chunk_body, 0)

  return embed_kernel


_EMBED = _make_kernel()


@jax.jit
def kernel(token_ids, table):
  ids = token_ids.reshape(-1).astype(jnp.int32)
  out = _EMBED(ids, table)
  return out.reshape(B, L, DIM)


# speedup vs baseline: 1.9055x; 1.1938x over previous
import functools

import jax
import jax.numpy as jnp
from jax import lax
from jax.experimental import pallas as pl
from jax.experimental.pallas import tpu as pltpu
from jax.experimental.pallas import tpu_sc as plsc

VOCAB = 1000000
DIM = 64
NUM_BITS = 16
BITS_BEGIN = 8
B = 16384
L = 50

N = B * L
NC, NS = 2, 16
NW = NC * NS
PER_W = N // NW
CHUNK = 128
NCHUNK = PER_W // CHUNK
NBUF = 4
GROUPS = NCHUNK // NBUF - 1

_DNUMS = lax.GatherDimensionNumbers(
    offset_dims=(), collapsed_slice_dims=(0,), start_index_map=(0,))


def _make_kernel():
  mesh = plsc.VectorSubcoreMesh(core_axis_name="c", subcore_axis_name="s")

  @functools.partial(
      pl.kernel,
      mesh=mesh,
      compiler_params=pltpu.CompilerParams(use_tc_tiling_on_sc=False),
      out_type=jax.ShapeDtypeStruct((N, DIM), jnp.float32),
      scratch_types=(
          [pltpu.VMEM((NCHUNK, CHUNK), jnp.int32)]
          + [pltpu.VMEM((CHUNK, DIM), jnp.float32) for _ in range(NBUF)]
          + [pltpu.SemaphoreType.DMA for _ in range(2 * NBUF)]
      ),
  )
  def embed_kernel(ids_hbm, table_hbm, out_hbm, idx2d, *rest):
    bufs = rest[:NBUF]
    gsems = rest[NBUF:2 * NBUF]
    osems = rest[2 * NBUF:]
    wid = lax.axis_index("s") * NC + lax.axis_index("c")
    w_base = wid * PER_W
    lanes = jnp.arange(NUM_BITS, dtype=jnp.int32)

    def start_gather(b, c):
      pltpu.async_copy(table_hbm.at[idx2d.at[c]], bufs[b], gsems[b])

    def wait_gather(b):
      pltpu.make_async_copy(
          table_hbm.at[idx2d.at[0]], bufs[b], gsems[b]).wait()

    def start_out(b, c):
      pltpu.async_copy(
          bufs[b], out_hbm.at[pl.ds(w_base + c * CHUNK, CHUNK)], osems[b])

    def wait_out(b):
      pltpu.make_async_copy(
          bufs[b], out_hbm.at[pl.ds(w_base, CHUNK)], osems[b]).wait()

    def patch(b, c):
      for g in range(CHUNK // 16):
        ids16 = idx2d[c, pl.ds(g * 16, 16)]
        for r in range(16):
          sel = jnp.full((16, 1), r, dtype=jnp.int32)
          idv = lax.gather(ids16, sel, _DNUMS, (1,),
                           mode=lax.GatherScatterMode.PROMISE_IN_BOUNDS)
          bits = ((idv >> lanes) & 1).astype(jnp.float32)
          bufs[b][g * 16 + r, pl.ds(BITS_BEGIN, NUM_BITS)] = bits

    pltpu.sync_copy(ids_hbm.at[pl.ds(wid * NCHUNK, NCHUNK)], idx2d)
    for b in range(NBUF):
      start_gather(b, b)

    def group_body(g, carry):
      for b in range(NBUF):
        c = g * NBUF + b
        wait_gather(b)
        patch(b, c)
        start_out(b, c)
        wait_out(b)
        start_gather(b, c + NBUF)
      return carry

    lax.fori_loop(0, GROUPS, group_body, 0)

    for b in range(NBUF):
      c = GROUPS * NBUF + b
      wait_gather(b)
      patch(b, c)
      start_out(b, c)
      wait_out(b)

  return embed_kernel


_EMBED = _make_kernel()


@jax.jit
def kernel(token_ids, table):
  ids = token_ids.reshape(NW * NCHUNK, CHUNK).astype(jnp.int32)
  out = _EMBED(ids, table)
  return out.reshape(B, L, DIM)
